# baseline (speedup 1.0000x reference)
import functools

import jax
import jax.numpy as jnp
from jax.experimental import pallas as pl
from jax.experimental.pallas import tpu as pltpu

N_CODES = 8
BLK = 1024


def _qinco_body(xh_ref, xt_ref, cb_ref, cand_ref, loss_ref):
    i = pl.program_id(0)
    r = xt_ref[...] - xh_ref[...]
    a2 = jnp.sum(r * r, axis=1, keepdims=True)
    cb = cb_ref[...]
    dots = jax.lax.dot_general(r, cb, (((1,), (1,)), ((), ())),
                               preferred_element_type=jnp.float32)
    b2 = jnp.sum(cb * cb, axis=1)[None, :]
    t = a2 + b2
    dists = t - 2.0 * dots

    K = dists.shape[1]
    iotaf = jax.lax.broadcasted_iota(jnp.int32, dists.shape, 1).astype(jnp.float32)
    loss_acc = None
    cols = []
    for j in range(N_CODES):
        m = jnp.min(dists, axis=1, keepdims=True)
        idxf = jnp.min(jnp.where(dists == m, iotaf, float(K)),
                       axis=1, keepdims=True)
        cols.append(idxf.astype(jnp.int32))
        if j == 0:
            loss_acc = jnp.sum(m)
        if j < N_CODES - 1:
            dists = jnp.where(iotaf == idxf, jnp.inf, dists)
    cand_ref[...] = jnp.concatenate(cols, axis=1)

    @pl.when(i == 0)
    def _init():
        loss_ref[...] = jnp.zeros((1, 1), jnp.float32)

    loss_ref[...] += jnp.full((1, 1), loss_acc, jnp.float32)


def kernel(xhat_BD, xtarget_BD, codebook_KD):
    B, d = xhat_BD.shape
    K = codebook_KD.shape[0]
    grid = (B // BLK,)
    cand_BA, loss_sum = pl.pallas_call(
        _qinco_body,
        grid=grid,
        in_specs=[
            pl.BlockSpec((BLK, d), lambda i: (i, 0)),
            pl.BlockSpec((BLK, d), lambda i: (i, 0)),
            pl.BlockSpec((K, d), lambda i: (0, 0)),
        ],
        out_specs=[
            pl.BlockSpec((BLK, N_CODES), lambda i: (i, 0)),
            pl.BlockSpec((1, 1), lambda i: (0, 0)),
        ],
        out_shape=[
            jax.ShapeDtypeStruct((B, N_CODES), jnp.int32),
            jax.ShapeDtypeStruct((1, 1), jnp.float32),
        ],
    )(xhat_BD, xtarget_BD, codebook_KD)

    loss = (loss_sum[0, 0] / (B * d)).astype(jnp.float32)
    return cand_BA, loss


# speedup vs baseline: 1.0145x; 1.0145x over previous
import functools

import jax
import jax.numpy as jnp
from jax.experimental import pallas as pl
from jax.experimental.pallas import tpu as pltpu

N_CODES = 8
BLK = 2048


def _qinco_body(xh_ref, xt_ref, cb_ref, b2_ref, cand_ref, loss_ref):
    i = pl.program_id(0)
    r = xt_ref[...] - xh_ref[...]
    a2 = jnp.sum(r * r, axis=1, keepdims=True)
    cb = cb_ref[...]
    dots = jax.lax.dot_general(r, cb, (((1,), (1,)), ((), ())),
                               preferred_element_type=jnp.float32)
    t = a2 + b2_ref[...]
    dists = t - 2.0 * dots

    K = dists.shape[1]
    iotaf = jax.lax.broadcasted_iota(jnp.int32, dists.shape, 1).astype(jnp.float32)
    loss_acc = None
    cols = []
    for j in range(N_CODES):
        m = jnp.min(dists, axis=1, keepdims=True)
        idxf = jnp.min(jnp.where(dists == m, iotaf, float(K)),
                       axis=1, keepdims=True)
        cols.append(idxf.astype(jnp.int32))
        if j == 0:
            loss_acc = jnp.sum(m)
        if j < N_CODES - 1:
            dists = jnp.where(iotaf == idxf, jnp.inf, dists)
    cand_ref[...] = jnp.concatenate(cols, axis=1)

    @pl.when(i == 0)
    def _init():
        loss_ref[...] = jnp.zeros((1, 1), jnp.float32)

    loss_ref[...] += jnp.full((1, 1), loss_acc, jnp.float32)


def kernel(xhat_BD, xtarget_BD, codebook_KD):
    B, d = xhat_BD.shape
    K = codebook_KD.shape[0]
    b2_K = jnp.sum(codebook_KD * codebook_KD, axis=1)[None, :]

    grid = (B // BLK,)
    cand_BA, loss_sum = pl.pallas_call(
        _qinco_body,
        grid=grid,
        in_specs=[
            pl.BlockSpec((BLK, d), lambda i: (i, 0)),
            pl.BlockSpec((BLK, d), lambda i: (i, 0)),
            pl.BlockSpec((K, d), lambda i: (0, 0)),
            pl.BlockSpec((1, K), lambda i: (0, 0)),
        ],
        out_specs=[
            pl.BlockSpec((BLK, N_CODES), lambda i: (i, 0)),
            pl.BlockSpec((1, 1), lambda i: (0, 0)),
        ],
        out_shape=[
            jax.ShapeDtypeStruct((B, N_CODES), jnp.int32),
            jax.ShapeDtypeStruct((1, 1), jnp.float32),
        ],
    )(xhat_BD, xtarget_BD, codebook_KD, b2_K)

    loss = (loss_sum[0, 0] / (B * d)).astype(jnp.float32)
    return cand_BA, loss
